# baseline (speedup 1.0000x reference)
import functools

import jax
import jax.numpy as jnp
from jax import lax
from jax.experimental import pallas as pl
from jax.experimental.pallas import tpu as pltpu
from jax.experimental.pallas import tpu_sc as plsc

NC = 2
NS = 16
CHUNK = 128


def _make_sc_agg(n, w, e):
    blk = 32
    assert e % (NC * NS * CHUNK * blk) == 0
    per_w = e // (NC * NS)
    n_iter = per_w // CHUNK
    n_blk = n_iter // blk
    assert n % (8 * NS) == 0
    rows_per_tile = n // NS

    mesh = plsc.VectorSubcoreMesh(core_axis_name="c", subcore_axis_name="s")

    @functools.partial(
        pl.kernel,
        out_type=jax.ShapeDtypeStruct((NC, n, w), jnp.float32),
        mesh=mesh,
        scratch_types=[
            pltpu.VMEM((blk, CHUNK), jnp.int32),
            pltpu.VMEM((blk, CHUNK), jnp.int32),
            pltpu.VMEM((blk, CHUNK), jnp.int32),
            pltpu.VMEM((blk, CHUNK), jnp.int32),
            pltpu.VMEM((CHUNK,), jnp.int32),
            pltpu.VMEM((CHUNK, w), jnp.float32),
            pltpu.VMEM((CHUNK, w), jnp.float32),
            pltpu.VMEM_SHARED((n, w), jnp.float32),
            pltpu.SemaphoreType.DMA,
            pltpu.SemaphoreType.DMA,
            pltpu.SemaphoreType.DMA,
        ],
    )
    def sc_agg(x_hbm, src_hbm, dst_hbm, out_hbm, sidx0, didx0, sidx1, didx1,
               dcur, rows0, rows1, acc, sem0, sem1, sem_s):

        def copy_idx_row(src2d, j, dst1d):
            for i in range(CHUNK // 16):
                dst1d[pl.ds(i * 16, 16)] = src2d[j, pl.ds(i * 16, 16)]

        c = lax.axis_index("c")
        s = lax.axis_index("s")
        wid = s * NC + c

        idx_bufs = [(sidx0, didx0), (sidx1, didx1)]

        def stage(b, sidx, didx):
            b0 = pl.multiple_of(wid * n_iter + b * blk, 8)
            pltpu.async_copy(src_hbm.at[pl.ds(b0, blk)], sidx, sem_s)
            pltpu.async_copy(dst_hbm.at[pl.ds(b0, blk)], didx, sem_s)

        def stage_wait(b, sidx, didx):
            b0 = pl.multiple_of(wid * n_iter + b * blk, 8)
            pltpu.make_async_copy(src_hbm.at[pl.ds(b0, blk)], sidx,
                                  sem_s).wait()
            pltpu.make_async_copy(dst_hbm.at[pl.ds(b0, blk)], didx,
                                  sem_s).wait()

        stage(0, sidx0, didx0)

        r0 = pl.multiple_of(s * rows_per_tile, 8)
        pltpu.sync_copy(x_hbm.at[pl.ds(r0, rows_per_tile)],
                        acc.at[pl.ds(r0, rows_per_tile)])
        plsc.subcore_barrier()

        for b in range(n_blk):
            sidx, didx = idx_bufs[b % 2]
            stage_wait(b, sidx, didx)
            if b + 1 < n_blk:
                stage(b + 1, *idx_bufs[(b + 1) % 2])

            pltpu.async_copy(x_hbm.at[sidx.at[0]], rows0, sem0)
            pltpu.async_copy(x_hbm.at[sidx.at[1]], rows1, sem1)

            def body(k, carry2, sidx=sidx, didx=didx):
                j0 = k * 2
                pltpu.make_async_copy(x_hbm.at[sidx.at[j0]], rows0,
                                      sem0).wait()
                copy_idx_row(didx, j0, dcur)
                pltpu.sync_copy(rows0, acc.at[dcur], add=True)

                @pl.when(j0 + 2 < blk)
                def _():
                    pltpu.async_copy(x_hbm.at[sidx.at[j0 + 2]], rows0, sem0)

                pltpu.make_async_copy(x_hbm.at[sidx.at[j0 + 1]], rows1,
                                      sem1).wait()
                copy_idx_row(didx, j0 + 1, dcur)
                pltpu.sync_copy(rows1, acc.at[dcur], add=True)

                @pl.when(j0 + 3 < blk)
                def _():
                    pltpu.async_copy(x_hbm.at[sidx.at[j0 + 3]], rows1, sem1)

                return carry2

            lax.fori_loop(0, blk // 2, body, 0)

        plsc.subcore_barrier()

        pltpu.sync_copy(acc.at[pl.ds(r0, rows_per_tile)],
                        out_hbm.at[c, pl.ds(r0, rows_per_tile)])

    return sc_agg


def _mlp_body(n, blk_r, p_ref, x_ref, w1_ref, b1_ref, w2_ref, b2_ref,
              h2_ref, stat_ref):
    step = pl.program_id(0)
    hid = p_ref[0] + p_ref[1] - x_ref[...]
    hid = jnp.dot(hid, w1_ref[...], preferred_element_type=jnp.float32)
    hid = jnp.maximum(hid + b1_ref[...], 0.0)
    hid = jnp.dot(hid, w2_ref[...], preferred_element_type=jnp.float32)
    hid = jnp.maximum(hid + b2_ref[...], 0.0)
    row = step * blk_r + lax.broadcasted_iota(jnp.int32, (blk_r, 1), 0)
    hid = jnp.where(row < n, hid, 0.0)
    h2_ref[...] = hid
    s1 = jnp.sum(hid, axis=0, keepdims=True)
    s2 = jnp.sum(hid * hid, axis=0, keepdims=True)

    @pl.when(step == 0)
    def _():
        stat_ref[...] = jnp.zeros_like(stat_ref)

    stat_ref[...] = stat_ref[...] + jnp.concatenate([s1, s2], axis=0)


def _pool_body(g, n, h2_ref, stat_ref, gamma_ref, beta_ref, batch_ref,
               lw_ref, lb_ref, xt_ref, pooled_ref):
    n_pad = h2_ref.shape[0]
    mean = stat_ref[0:1, :] / n
    var = stat_ref[1:2, :] / n - mean * mean
    inv = lax.rsqrt(var + 1e-5)
    hid = (h2_ref[...] - mean) * (inv * gamma_ref[...]) + beta_ref[...]
    hid = jnp.maximum(hid, 0.0)
    onehot = (batch_ref[...] ==
              lax.broadcasted_iota(jnp.int32, (1, g), 1)).astype(jnp.float32)
    sums = lax.dot_general(onehot, hid, (((0,), (0,)), ((), ())),
                           preferred_element_type=jnp.float32)
    ones = jnp.ones((n_pad, 1), jnp.float32)
    counts = lax.dot_general(onehot, ones, (((0,), (0,)), ((), ())),
                             preferred_element_type=jnp.float32)
    pooled = sums / jnp.maximum(counts, 1.0)
    xt = jnp.dot(pooled, lw_ref[...],
                 preferred_element_type=jnp.float32) + lb_ref[...]
    xt_ref[...] = xt
    pooled_ref[...] = pooled


def kernel(x, edge_index, batch, W1, b1, W2, b2, gamma, beta, lin_w, lin_b):
    n, f_in = x.shape
    h = W1.shape[1]
    e = edge_index.shape[1]
    g = 86
    w = 128

    n_pad = ((n + 8 * NS - 1) // (8 * NS)) * (8 * NS)
    x_pad = jnp.pad(x, ((0, n_pad - n), (0, w - f_in)))
    batch_pad = jnp.pad(batch, (0, n_pad - n), constant_values=g)

    e_unit = 32 * NC * NS * CHUNK
    e_pad = ((e + e_unit - 1) // e_unit) * e_unit
    src = edge_index[0]
    dst = edge_index[1]
    if e_pad != e:
        fill = n + jnp.arange(e_pad - e, dtype=jnp.int32) % (n_pad - n)
        src = jnp.concatenate([src, fill])
        dst = jnp.concatenate([dst, fill])

    sc_agg = _make_sc_agg(n_pad, w, e_pad)
    p = sc_agg(x_pad, src.reshape(-1, CHUNK), dst.reshape(-1, CHUNK))

    w1_pad = jnp.pad(W1, ((0, w - f_in), (0, 0)))
    blk_r = 1264
    h2, stat = pl.pallas_call(
        functools.partial(_mlp_body, n, blk_r),
        grid=(n_pad // blk_r,),
        in_specs=[
            pl.BlockSpec((2, blk_r, w), lambda i: (0, i, 0)),
            pl.BlockSpec((blk_r, w), lambda i: (i, 0)),
            pl.BlockSpec((w, h), lambda i: (0, 0)),
            pl.BlockSpec((1, h), lambda i: (0, 0)),
            pl.BlockSpec((h, h), lambda i: (0, 0)),
            pl.BlockSpec((1, h), lambda i: (0, 0)),
        ],
        out_specs=[
            pl.BlockSpec((blk_r, h), lambda i: (i, 0)),
            pl.BlockSpec((2, h), lambda i: (0, 0)),
        ],
        out_shape=(
            jax.ShapeDtypeStruct((n_pad, h), jnp.float32),
            jax.ShapeDtypeStruct((2, h), jnp.float32),
        ),
    )(p, x_pad, w1_pad, b1.reshape(1, h), W2, b2.reshape(1, h))

    xt, pooled = pl.pallas_call(
        functools.partial(_pool_body, g, n),
        out_shape=(
            jax.ShapeDtypeStruct((g, lin_w.shape[1]), jnp.float32),
            jax.ShapeDtypeStruct((g, h), jnp.float32),
        ),
    )(h2, stat, gamma.reshape(1, h), beta.reshape(1, h),
      batch_pad.reshape(n_pad, 1), lin_w, lin_b.reshape(1, lin_w.shape[1]))
    return (xt, pooled)


# speedup vs baseline: 1.0284x; 1.0284x over previous
import functools

import jax
import jax.numpy as jnp
from jax import lax
from jax.experimental import pallas as pl
from jax.experimental.pallas import tpu as pltpu
from jax.experimental.pallas import tpu_sc as plsc

NC = 2
NS = 16
CHUNK = 128


def _make_sc_agg(n, w, e):
    blk = 32
    assert e % (NC * NS * CHUNK * blk) == 0
    per_w = e // (NC * NS)
    n_iter = per_w // CHUNK
    n_blk = n_iter // blk
    assert n % (8 * NS) == 0
    rows_per_tile = n // NS

    mesh = plsc.VectorSubcoreMesh(core_axis_name="c", subcore_axis_name="s")

    @functools.partial(
        pl.kernel,
        out_type=jax.ShapeDtypeStruct((NC, n, w), jnp.float32),
        mesh=mesh,
        scratch_types=[
            pltpu.VMEM((blk, CHUNK), jnp.int32),
            pltpu.VMEM((blk, CHUNK), jnp.int32),
            pltpu.VMEM((blk, CHUNK), jnp.int32),
            pltpu.VMEM((blk, CHUNK), jnp.int32),
            pltpu.VMEM((CHUNK,), jnp.int32),
            pltpu.VMEM((CHUNK, w), jnp.float32),
            pltpu.VMEM((CHUNK, w), jnp.float32),
            pltpu.VMEM_SHARED((n, w), jnp.float32),
            pltpu.SemaphoreType.DMA,
            pltpu.SemaphoreType.DMA,
            pltpu.SemaphoreType.DMA,
        ],
    )
    def sc_agg(x_hbm, src_hbm, dst_hbm, out_hbm, sidx0, didx0, sidx1, didx1,
               dcur, rows0, rows1, acc, sem0, sem1, sem_s):

        def copy_idx_row(src2d, j, dst1d):
            for i in range(CHUNK // 16):
                dst1d[pl.ds(i * 16, 16)] = src2d[j, pl.ds(i * 16, 16)]

        c = lax.axis_index("c")
        s = lax.axis_index("s")
        wid = s * NC + c

        idx_bufs = [(sidx0, didx0), (sidx1, didx1)]

        def stage(b, sidx, didx):
            b0 = pl.multiple_of(wid * n_iter + b * blk, 8)
            pltpu.async_copy(src_hbm.at[pl.ds(b0, blk)], sidx, sem_s)
            pltpu.async_copy(dst_hbm.at[pl.ds(b0, blk)], didx, sem_s)

        def stage_wait(b, sidx, didx):
            b0 = pl.multiple_of(wid * n_iter + b * blk, 8)
            pltpu.make_async_copy(src_hbm.at[pl.ds(b0, blk)], sidx,
                                  sem_s).wait()
            pltpu.make_async_copy(dst_hbm.at[pl.ds(b0, blk)], didx,
                                  sem_s).wait()

        stage(0, sidx0, didx0)

        r0 = pl.multiple_of(s * rows_per_tile, 8)
        pltpu.sync_copy(x_hbm.at[pl.ds(r0, rows_per_tile)],
                        acc.at[pl.ds(r0, rows_per_tile)])
        plsc.subcore_barrier()

        for b in range(n_blk):
            sidx, didx = idx_bufs[b % 2]
            stage_wait(b, sidx, didx)
            if b + 1 < n_blk:
                stage(b + 1, *idx_bufs[(b + 1) % 2])

            pltpu.async_copy(x_hbm.at[sidx.at[0]], rows0, sem0)
            pltpu.async_copy(x_hbm.at[sidx.at[1]], rows1, sem1)

            def body(k, carry2, sidx=sidx, didx=didx):
                j0 = k * 2
                pltpu.make_async_copy(x_hbm.at[sidx.at[j0]], rows0,
                                      sem0).wait()
                copy_idx_row(didx, j0, dcur)
                pltpu.sync_copy(rows0, acc.at[dcur], add=True)

                @pl.when(j0 + 2 < blk)
                def _():
                    pltpu.async_copy(x_hbm.at[sidx.at[j0 + 2]], rows0, sem0)

                pltpu.make_async_copy(x_hbm.at[sidx.at[j0 + 1]], rows1,
                                      sem1).wait()
                copy_idx_row(didx, j0 + 1, dcur)
                pltpu.sync_copy(rows1, acc.at[dcur], add=True)

                @pl.when(j0 + 3 < blk)
                def _():
                    pltpu.async_copy(x_hbm.at[sidx.at[j0 + 3]], rows1, sem1)

                return carry2

            lax.fori_loop(0, blk // 2, body, 0)

        plsc.subcore_barrier()

        pltpu.sync_copy(acc.at[pl.ds(r0, rows_per_tile)],
                        out_hbm.at[c, pl.ds(r0, rows_per_tile)])

    return sc_agg


def _tail_body(g, n, p_ref, x_ref, w1_ref, b1_ref, w2_ref, b2_ref, gamma_ref,
               beta_ref, batch_ref, lw_ref, lb_ref, xt_ref, pooled_ref):
    n_pad = x_ref.shape[0]
    hid = p_ref[0] + p_ref[1] - x_ref[...]
    hid = jnp.dot(hid, w1_ref[...], preferred_element_type=jnp.float32)
    hid = jnp.maximum(hid + b1_ref[...], 0.0)
    hid = jnp.dot(hid, w2_ref[...], preferred_element_type=jnp.float32)
    hid = jnp.maximum(hid + b2_ref[...], 0.0)
    row_ok = lax.broadcasted_iota(jnp.int32, (n_pad, 1), 0) < n
    hid = jnp.where(row_ok, hid, 0.0)
    s1 = jnp.sum(hid, axis=0, keepdims=True)
    s2 = jnp.sum(hid * hid, axis=0, keepdims=True)
    mean = s1 / n
    var = s2 / n - mean * mean
    inv = lax.rsqrt(var + 1e-5)
    hid = (hid - mean) * (inv * gamma_ref[...]) + beta_ref[...]
    hid = jnp.maximum(hid, 0.0)
    onehot = (batch_ref[...] ==
              lax.broadcasted_iota(jnp.int32, (1, g), 1)).astype(jnp.float32)
    sums = lax.dot_general(onehot, hid, (((0,), (0,)), ((), ())),
                           preferred_element_type=jnp.float32)
    ones = jnp.ones((n_pad, 1), jnp.float32)
    counts = lax.dot_general(onehot, ones, (((0,), (0,)), ((), ())),
                             preferred_element_type=jnp.float32)
    pooled = sums / jnp.maximum(counts, 1.0)
    xt = jnp.dot(pooled, lw_ref[...],
                 preferred_element_type=jnp.float32) + lb_ref[...]
    xt_ref[...] = xt
    pooled_ref[...] = pooled


def kernel(x, edge_index, batch, W1, b1, W2, b2, gamma, beta, lin_w, lin_b):
    n, f_in = x.shape
    h = W1.shape[1]
    e = edge_index.shape[1]
    g = 86
    w = 128

    n_pad = ((n + 8 * NS - 1) // (8 * NS)) * (8 * NS)
    x_pad = jnp.pad(x, ((0, n_pad - n), (0, w - f_in)))
    batch_pad = jnp.pad(batch, (0, n_pad - n), constant_values=g)

    e_unit = 32 * NC * NS * CHUNK
    e_pad = ((e + e_unit - 1) // e_unit) * e_unit
    src = edge_index[0]
    dst = edge_index[1]
    if e_pad != e:
        fill = n + jnp.arange(e_pad - e, dtype=jnp.int32) % (n_pad - n)
        src = jnp.concatenate([src, fill])
        dst = jnp.concatenate([dst, fill])

    sc_agg = _make_sc_agg(n_pad, w, e_pad)
    p = sc_agg(x_pad, src.reshape(-1, CHUNK), dst.reshape(-1, CHUNK))

    w1_pad = jnp.pad(W1, ((0, w - f_in), (0, 0)))
    xt, pooled = pl.pallas_call(
        functools.partial(_tail_body, g, n),
        out_shape=(
            jax.ShapeDtypeStruct((g, lin_w.shape[1]), jnp.float32),
            jax.ShapeDtypeStruct((g, h), jnp.float32),
        ),
    )(p, x_pad, w1_pad, b1.reshape(1, h), W2, b2.reshape(1, h),
      gamma.reshape(1, h), beta.reshape(1, h), batch_pad.reshape(n_pad, 1),
      lin_w, lin_b.reshape(1, lin_w.shape[1]))
    return (xt, pooled)
